# Initial kernel scaffold; baseline (speedup 1.0000x reference)
import jax
import jax.numpy as jnp
from jax.experimental import pallas as pl


def kernel(x, prefix_len, dropout_mask_token):
    raise NotImplementedError("write your pallas kernel here")


# speedup vs baseline: 1.1114x; 1.1114x over previous
import jax
import jax.numpy as jnp
from jax.experimental import pallas as pl
from jax.experimental.pallas import tpu as pltpu

_B, _T, _S, _D = 8, 16, 576, 768


def _body(pref, x_ref, tok_ref, o_ref):
    b = pl.program_id(0)
    t = pl.program_id(1)
    masked = t < pref[b]

    @pl.when(masked)
    def _():
        o_ref[...] = jnp.broadcast_to(tok_ref[...][None, None, :, :], (1, 1, _S, _D))

    @pl.when(jnp.logical_not(masked))
    def _():
        o_ref[...] = x_ref[...]


def _x_index_map(b, t, pref):
    p = pref[b]
    t_in = jnp.where(t < p, jnp.minimum(p, _T - 1), t)
    return b, t_in, 0, 0


def kernel(x, prefix_len, dropout_mask_token):
    tok2d = dropout_mask_token.reshape(1, _D)
    grid_spec = pltpu.PrefetchScalarGridSpec(
        num_scalar_prefetch=1,
        grid=(_B, _T),
        in_specs=[
            pl.BlockSpec((1, 1, _S, _D), _x_index_map),
            pl.BlockSpec((1, _D), lambda b, t, pref: (0, 0)),
        ],
        out_specs=pl.BlockSpec((1, 1, _S, _D), lambda b, t, pref: (b, t, 0, 0)),
    )
    fn = pl.pallas_call(
        _body,
        grid_spec=grid_spec,
        out_shape=jax.ShapeDtypeStruct(x.shape, x.dtype),
    )
    return fn(prefix_len, x, tok2d)
